# baseline (device time: 150723 ns/iter reference)
import jax
import jax.numpy as jnp
from jax import lax
from jax.experimental import pallas as pl
from jax.experimental.pallas import tpu as pltpu

N_DEV = 4
HQ = 8
DH = 128
SCALE = 0.08838834764831843
EXP_OFF = 8.0


def kernel(x, Wq, Wo, K_ext, V_ext):
    Sq = x.shape[1]
    D = x.shape[2]
    Skv = K_ext.shape[1]

    kb = K_ext[0].reshape(Skv, D).astype(jnp.bfloat16)
    vb = V_ext[0].reshape(Skv, D).astype(jnp.bfloat16)

    def body(x_ref, wq_ref, wo_ref, k_ref, v_ref, out_ref,
             q_buf, acc_buf, st_buf, attn_scr,
             q_send, q_recv, a_send, a_recv, s_send, s_recv):
        my = lax.axis_index("i")
        right = lax.rem(my + 1, N_DEV)
        left = lax.rem(my + N_DEV - 1, N_DEV)

        barrier = pltpu.get_barrier_semaphore()
        for nbr in (left, right):
            pl.semaphore_signal(barrier, inc=1, device_id=(nbr,),
                                device_id_type=pl.DeviceIdType.MESH)
        pl.semaphore_wait(barrier, 2)

        def q_rdma(src_slot, dst_slot, dev):
            return pltpu.make_async_remote_copy(
                src_ref=q_buf.at[src_slot],
                dst_ref=q_buf.at[dst_slot],
                send_sem=q_send.at[src_slot],
                recv_sem=q_recv.at[dst_slot],
                device_id=(dev,),
                device_id_type=pl.DeviceIdType.MESH,
            )

        def head_rdma(buf, ss, rs, src_slot, dst_slot, h, dev):
            return pltpu.make_async_remote_copy(
                src_ref=buf.at[src_slot, h],
                dst_ref=buf.at[dst_slot, h],
                send_sem=ss.at[src_slot, h],
                recv_sem=rs.at[dst_slot, h],
                device_id=(dev,),
                device_id_type=pl.DeviceIdType.MESH,
            )

        def flash_pair(slot, h0, first):
            h1 = h0 + 1

            def qk(h):
                qh = q_buf[slot, :, h * DH:(h + 1) * DH]
                return lax.dot_general(
                    qh, k_ref[:, h * DH:(h + 1) * DH],
                    (((1,), (1,)), ((), ())),
                    preferred_element_type=jnp.float32)

            def pv_dot(p, h):
                return lax.dot_general(
                    p, v_ref[:, h * DH:(h + 1) * DH],
                    (((1,), (0,)), ((), ())),
                    preferred_element_type=jnp.float32)

            s0 = qk(h0)
            s1 = qk(h1)
            p0 = jnp.exp(s0 - EXP_OFF)
            b0 = p0.astype(jnp.bfloat16)
            pv0 = pv_dot(b0, h0)
            p1 = jnp.exp(s1 - EXP_OFF)
            ps0 = jnp.sum(p0, axis=1, keepdims=True)
            b1 = p1.astype(jnp.bfloat16)
            pv1 = pv_dot(b1, h1)
            ps1 = jnp.sum(p1, axis=1, keepdims=True)
            for h, pv, ps in ((h0, pv0, ps0), (h1, pv1, ps1)):
                if first:
                    acc_buf[slot, h] = pv
                    st_buf[slot, h, :, 0:1] = ps
                else:
                    acc_buf[slot, h] = acc_buf[slot, h] + pv
                    st_buf[slot, h, :, 0:1] = st_buf[slot, h, :, 0:1] + ps

        def send_head(step, h):
            dst = (step + 1) % N_DEV
            head_rdma(acc_buf, a_send, a_recv, step, dst, h, right).start()
            head_rdma(st_buf, s_send, s_recv, step, dst, h, right).start()

        def wait_recv_head(slot, h):
            head_rdma(acc_buf, a_send, a_recv, slot, slot, h, left).wait_recv()
            head_rdma(st_buf, s_send, s_recv, slot, slot, h, left).wait_recv()

        q = lax.dot_general(
            x_ref[:, :], wq_ref[:, :], (((1,), (0,)), ((), ())),
            preferred_element_type=jnp.float32)
        q_buf[0, :, :] = (q * SCALE).astype(jnp.bfloat16)
        q_rdma(0, 1, right).start()
        for hp in range(0, HQ, 2):
            flash_pair(0, hp, first=True)
            send_head(0, hp)
            send_head(0, hp + 1)

        for step in (1, 2, 3):
            q_rdma(step, step, left).wait_recv()
            if step < 3:
                q_rdma(step, step + 1, right).start()
            for hp in range(0, HQ, 2):
                wait_recv_head(step, hp)
                wait_recv_head(step, hp + 1)
                flash_pair(step, hp, first=False)
                send_head(step, hp)
                send_head(step, hp + 1)

        for h in range(HQ):
            wait_recv_head(0, h)
            l = st_buf[0, h, :, 0:1]
            attn_scr[:, h * DH:(h + 1) * DH] = (
                acc_buf[0, h] / l).astype(jnp.bfloat16)
        out_ref[:, :] = lax.dot_general(
            attn_scr[:, :], wo_ref[:, :], (((1,), (0,)), ((), ())),
            preferred_element_type=jnp.float32)

        for step in range(N_DEV):
            dst = (step + 1) % N_DEV
            if step < 3:
                q_rdma(step, dst, right).wait_send()
            for h in range(HQ):
                head_rdma(acc_buf, a_send, a_recv, step, dst, h,
                          right).wait_send()
                head_rdma(st_buf, s_send, s_recv, step, dst, h,
                          right).wait_send()

    out = pl.pallas_call(
        body,
        out_shape=jax.ShapeDtypeStruct((Sq, D), jnp.float32),
        in_specs=[pl.BlockSpec(memory_space=pltpu.VMEM)] * 5,
        out_specs=pl.BlockSpec(memory_space=pltpu.VMEM),
        scratch_shapes=[
            pltpu.VMEM((N_DEV, Sq, D), jnp.bfloat16),
            pltpu.VMEM((N_DEV, HQ, Sq, DH), jnp.float32),
            pltpu.VMEM((N_DEV, HQ, Sq, 1), jnp.float32),
            pltpu.VMEM((Sq, D), jnp.bfloat16),
            pltpu.SemaphoreType.DMA((N_DEV,)),
            pltpu.SemaphoreType.DMA((N_DEV,)),
            pltpu.SemaphoreType.DMA((N_DEV, HQ)),
            pltpu.SemaphoreType.DMA((N_DEV, HQ)),
            pltpu.SemaphoreType.DMA((N_DEV, HQ)),
            pltpu.SemaphoreType.DMA((N_DEV, HQ)),
        ],
        compiler_params=pltpu.CompilerParams(
            collective_id=0, vmem_limit_bytes=100 * 1024 * 1024),
    )(x[0].astype(jnp.bfloat16), Wq.astype(jnp.bfloat16),
      Wo.astype(jnp.bfloat16), kb, vb)

    return out.reshape(1, Sq, D)
